# baseline (speedup 1.0000x reference)
import functools

import jax
import jax.numpy as jnp
from jax import lax
from jax.experimental import pallas as pl
from jax.experimental.pallas import tpu as pltpu
from jax.experimental.pallas import tpu_sc as plsc

F = 26
V = 100000
D = 32
B = 16384
N = B * F
NC, NS, L = 2, 16, 16
NW = NC * NS
N_PER_W = N // NW
VECS = N_PER_W // L
CHUNK = 128
N_CHUNKS = N_PER_W // CHUNK


def _sc_gather(x_flat, table_flat):
  mesh = plsc.VectorSubcoreMesh(core_axis_name="c", subcore_axis_name="s")

  @functools.partial(
      pl.kernel,
      out_type=jax.ShapeDtypeStruct((N, D), jnp.float32),
      mesh=mesh,
      scratch_types=[
          pltpu.VMEM((N_PER_W,), jnp.int32),
          pltpu.VMEM((CHUNK, D), jnp.float32),
          pltpu.SemaphoreType.DMA,
      ],
      compiler_params=pltpu.CompilerParams(use_tc_tiling_on_sc=False),
  )
  def k(x_hbm, tab_hbm, out_hbm, idx_v, rows_v, sem):
    wid = lax.axis_index("s") * NC + lax.axis_index("c")
    base = wid * N_PER_W

    pltpu.sync_copy(x_hbm.at[pl.ds(base, N_PER_W)], idx_v)

    def vbody(i, carry):
      off = i * L
      pos = off + lax.iota(jnp.int32, L)
      fld = lax.rem(pos, F)
      idx_v[pl.ds(off, L)] = idx_v[pl.ds(off, L)] + fld * V
      return carry

    lax.fori_loop(0, VECS, vbody, 0)

    def cbody(c, carry):
      start = c * CHUNK
      cp = pltpu.async_copy(
          tab_hbm.at[idx_v.at[pl.ds(start, CHUNK)]], rows_v, sem)
      cp.wait()
      pltpu.sync_copy(rows_v, out_hbm.at[pl.ds(base + start, CHUNK)])
      return carry

    lax.fori_loop(0, N_CHUNKS, cbody, 0)

  return k(x_flat, table_flat)


def kernel(x_cat, tables):
  out = _sc_gather(x_cat.reshape(N), tables.reshape(F * V, D))
  return out.reshape(B, F, D)


# speedup vs baseline: 1.0547x; 1.0547x over previous
import functools

import jax
import jax.numpy as jnp
from jax import lax
from jax.experimental import pallas as pl
from jax.experimental.pallas import tpu as pltpu
from jax.experimental.pallas import tpu_sc as plsc

F = 26
V = 100000
D = 32
B = 16384
N = B * F
NC, NS, L = 2, 16, 16
NW = NC * NS
N_PER_W = N // NW
UNROLL = 8
VEC_ITERS = N_PER_W // (L * UNROLL)
CHUNK = 832
N_CHUNKS = N_PER_W // CHUNK
N_PAIRS = N_CHUNKS // 2


def _sc_gather(x_flat, table_flat):
  mesh = plsc.VectorSubcoreMesh(core_axis_name="c", subcore_axis_name="s")

  @functools.partial(
      pl.kernel,
      out_type=jax.ShapeDtypeStruct((N, D), jnp.float32),
      mesh=mesh,
      scratch_types=[
          pltpu.VMEM((N_PER_W,), jnp.int32),
          pltpu.VMEM((CHUNK, D), jnp.float32),
          pltpu.VMEM((CHUNK, D), jnp.float32),
          pltpu.SemaphoreType.DMA,
          pltpu.SemaphoreType.DMA,
      ],
      compiler_params=pltpu.CompilerParams(use_tc_tiling_on_sc=False),
  )
  def k(x_hbm, tab_hbm, out_hbm, idx_v, rows0, rows1, sem0, sem1):
    wid = lax.axis_index("s") * NC + lax.axis_index("c")
    base = wid * N_PER_W

    pltpu.sync_copy(x_hbm.at[pl.ds(base, N_PER_W)], idx_v)

    lanes = lax.iota(jnp.int32, L)

    def vbody(i, carry):
      for j in range(UNROLL):
        off = (i * UNROLL + j) * L
        fld = lax.rem(off + lanes, F)
        idx_v[pl.ds(off, L)] = idx_v[pl.ds(off, L)] + fld * V
      return carry

    lax.fori_loop(0, VEC_ITERS, vbody, 0)

    def gather(c, buf, sem):
      return pltpu.async_copy(
          tab_hbm.at[idx_v.at[pl.ds(c * CHUNK, CHUNK)]], buf, sem)

    def wait(buf, sem):
      pltpu.make_async_copy(
          tab_hbm.at[idx_v.at[pl.ds(0, CHUNK)]], buf, sem).wait()

    def writeout(c, buf):
      pltpu.sync_copy(buf, out_hbm.at[pl.ds(base + c * CHUNK, CHUNK)])

    gather(0, rows0, sem0)

    def pbody(g, carry):
      c0 = 2 * g
      gather(c0 + 1, rows1, sem1)
      wait(rows0, sem0)
      writeout(c0, rows0)
      nxt = lax.rem(c0 + 2, N_CHUNKS)
      gather(nxt, rows0, sem0)
      wait(rows1, sem1)
      writeout(c0 + 1, rows1)
      return carry

    lax.fori_loop(0, N_PAIRS, pbody, 0)
    wait(rows0, sem0)

  return k(x_flat, table_flat)


def kernel(x_cat, tables):
  out = _sc_gather(x_cat.reshape(N), tables.reshape(F * V, D))
  return out.reshape(B, F, D)
